# baseline (speedup 1.0000x reference)
import math

import jax
import jax.numpy as jnp
from jax.experimental import pallas as pl
from jax.experimental.pallas import tpu as pltpu

B = 128
M = 1024
K = 128
V = 512

CB = 4
NBUF = 4
NCHUNK = B // CB


def _episodic_kernel(hidden_ref, keys_hbm, values_hbm, filled_ref,
                     wq_ref, bq_ref, w1h_ref, w1r_ref, b1_ref,
                     w2_ref, b2_ref, wo_ref, bo_ref, out_ref,
                     kbuf, vbuf, ksem, vsem):
    scale = 1.0 / math.sqrt(K)

    def start(i):
        slot = i % NBUF
        pltpu.make_async_copy(keys_hbm.at[i], kbuf.at[slot], ksem.at[slot]).start()
        pltpu.make_async_copy(values_hbm.at[i], vbuf.at[slot], vsem.at[slot]).start()

    def wait(i):
        slot = i % NBUF
        pltpu.make_async_copy(keys_hbm.at[i], kbuf.at[slot], ksem.at[slot]).wait()
        pltpu.make_async_copy(values_hbm.at[i], vbuf.at[slot], vsem.at[slot]).wait()

    for j in range(min(NBUF, NCHUNK)):
        start(j)

    for i in range(NCHUNK):
        wait(i)
        slot = i % NBUF
        h = hidden_ref[i]
        q = jax.lax.dot_general(h, wq_ref[...], (((1,), (1,)), ((), ())),
                                preferred_element_type=jnp.float32) + bq_ref[...]
        scores = jnp.concatenate([
            jax.lax.dot_general(q[j:j + 1], kbuf[slot, j],
                                (((1,), (1,)), ((), ())),
                                preferred_element_type=jnp.float32)
            for j in range(CB)], axis=0)
        slot_ids = jax.lax.broadcasted_iota(jnp.int32, (CB, M), 1)
        valid = slot_ids < filled_ref[i]
        scores = jnp.where(valid, scores * scale, -jnp.inf)
        m = jnp.max(scores, axis=-1, keepdims=True)
        m = jnp.where(jnp.isfinite(m), m, 0.0)
        e = jnp.exp(scores - m)
        s = jnp.sum(e, axis=-1, keepdims=True)
        attn = jnp.where(s > 0.0, e / s, 0.0)
        retrieved = jnp.concatenate([
            jax.lax.dot_general(attn[j:j + 1], vbuf[slot, j],
                                (((1,), (0,)), ((), ())),
                                preferred_element_type=jnp.float32)
            for j in range(CB)], axis=0)
        if i + NBUF < NCHUNK:
            start(i + NBUF)
        g = (jax.lax.dot_general(h, w1h_ref[...], (((1,), (1,)), ((), ())),
                                 preferred_element_type=jnp.float32)
             + jax.lax.dot_general(retrieved, w1r_ref[...], (((1,), (1,)), ((), ())),
                                   preferred_element_type=jnp.float32)
             + b1_ref[...])
        h1 = g * jax.nn.sigmoid(g)
        gate = jax.nn.sigmoid(
            jax.lax.dot_general(h1, w2_ref[...], (((1,), (1,)), ((), ())),
                                preferred_element_type=jnp.float32) + b2_ref[...])
        y = h + gate * retrieved
        out_ref[i] = jax.lax.dot_general(y, wo_ref[...], (((1,), (1,)), ((), ())),
                                         preferred_element_type=jnp.float32) + bo_ref[...]


def kernel(hidden, mem_keys, mem_values, Wq, bq, W1, b1, W2, b2, Wo, bo, filled):
    hidden3 = hidden.reshape(NCHUNK, CB, V)
    filled3 = filled.astype(jnp.int32).reshape(NCHUNK, CB, 1)
    keys4 = mem_keys.reshape(NCHUNK, CB, M, K)
    values4 = mem_values.reshape(NCHUNK, CB, M, V)
    W1h = W1[:, :V]
    W1r = W1[:, V:]

    vmem = pl.BlockSpec(memory_space=pltpu.MemorySpace.VMEM)
    hbm = pl.BlockSpec(memory_space=pltpu.MemorySpace.HBM)
    out = pl.pallas_call(
        _episodic_kernel,
        in_specs=[vmem, hbm, hbm, vmem,
                  vmem, vmem, vmem, vmem, vmem, vmem, vmem, vmem, vmem],
        out_specs=vmem,
        out_shape=jax.ShapeDtypeStruct((NCHUNK, CB, V), jnp.float32),
        scratch_shapes=[
            pltpu.VMEM((NBUF, CB, M, K), jnp.float32),
            pltpu.VMEM((NBUF, CB, M, V), jnp.float32),
            pltpu.SemaphoreType.DMA((NBUF,)),
            pltpu.SemaphoreType.DMA((NBUF,)),
        ],
    )(hidden3, keys4, values4, filled3,
      Wq, bq.reshape(1, K), W1h, W1r, b1.reshape(1, V),
      W2, b2.reshape(1, V), Wo, bo.reshape(1, V))
    return out.reshape(B, V)


# speedup vs baseline: 1.0244x; 1.0244x over previous
import math

import jax
import jax.numpy as jnp
from jax.experimental import pallas as pl
from jax.experimental.pallas import tpu as pltpu

B = 128
M = 1024
K = 128
V = 512

BB = 8
VH = V // 2


def _episodic_kernel(hidden_ref, keys_ref, values_ref, filled_ref,
                     wq_ref, bq_ref, w1h_ref, w1r_ref, b1_ref,
                     w2_ref, b2_ref, wo_ref, bo_ref, out_ref,
                     attn_s, retr_s):
    scale = 1.0 / math.sqrt(K)
    v = pl.program_id(1)
    h = hidden_ref[0]

    @pl.when(v == 0)
    def _scores():
        q = jax.lax.dot_general(h, wq_ref[...], (((1,), (1,)), ((), ())),
                                preferred_element_type=jnp.float32) + bq_ref[...]
        scores = jnp.concatenate([
            jax.lax.dot_general(q[j:j + 1], keys_ref[0, j],
                                (((1,), (1,)), ((), ())),
                                preferred_element_type=jnp.float32)
            for j in range(BB)], axis=0)
        slot = jax.lax.broadcasted_iota(jnp.int32, (BB, M), 1)
        valid = slot < filled_ref[0]
        scores = jnp.where(valid, scores * scale, -jnp.inf)
        m = jnp.max(scores, axis=-1, keepdims=True)
        m = jnp.where(jnp.isfinite(m), m, 0.0)
        e = jnp.exp(scores - m)
        s = jnp.sum(e, axis=-1, keepdims=True)
        attn = jnp.where(s > 0.0, e / s, 0.0)
        attn_s[...] = attn
        retr_s[...] = jnp.concatenate([
            jax.lax.dot_general(attn[j:j + 1], values_ref[0, j],
                                (((1,), (0,)), ((), ())),
                                preferred_element_type=jnp.float32)
            for j in range(BB)], axis=0)

    @pl.when(v == 1)
    def _finish():
        attn = attn_s[...]
        retr1 = jnp.concatenate([
            jax.lax.dot_general(attn[j:j + 1], values_ref[0, j],
                                (((1,), (0,)), ((), ())),
                                preferred_element_type=jnp.float32)
            for j in range(BB)], axis=0)
        retrieved = jnp.concatenate([retr_s[...], retr1], axis=1)
        g = (jax.lax.dot_general(h, w1h_ref[...], (((1,), (1,)), ((), ())),
                                 preferred_element_type=jnp.float32)
             + jax.lax.dot_general(retrieved, w1r_ref[...], (((1,), (1,)), ((), ())),
                                   preferred_element_type=jnp.float32)
             + b1_ref[...])
        h1 = g * jax.nn.sigmoid(g)
        gate = jax.nn.sigmoid(
            jax.lax.dot_general(h1, w2_ref[...], (((1,), (1,)), ((), ())),
                                preferred_element_type=jnp.float32) + b2_ref[...])
        y = h + gate * retrieved
        out_ref[0] = jax.lax.dot_general(y, wo_ref[...], (((1,), (1,)), ((), ())),
                                         preferred_element_type=jnp.float32) + bo_ref[...]


def kernel(hidden, mem_keys, mem_values, Wq, bq, W1, b1, W2, b2, Wo, bo, filled):
    nb = B // BB
    hidden3 = hidden.reshape(nb, BB, V)
    filled3 = filled.astype(jnp.int32).reshape(nb, BB, 1)
    keys4 = mem_keys.reshape(nb, BB, M, K)
    values4 = mem_values.reshape(nb, BB, M, V)
    W1h = W1[:, :V]
    W1r = W1[:, V:]
    rep2 = lambda i, v: (0, 0)

    out = pl.pallas_call(
        _episodic_kernel,
        grid=(nb, 2),
        in_specs=[
            pl.BlockSpec((1, BB, V), lambda i, v: (i, 0, 0)),
            pl.BlockSpec((1, BB, M, K), lambda i, v: (i, 0, 0, 0)),
            pl.BlockSpec((1, BB, M, VH), lambda i, v: (i, 0, 0, v)),
            pl.BlockSpec((1, BB, 1), lambda i, v: (i, 0, 0)),
            pl.BlockSpec((K, V), rep2),
            pl.BlockSpec((1, K), rep2),
            pl.BlockSpec((V, V), rep2),
            pl.BlockSpec((V, V), rep2),
            pl.BlockSpec((1, V), rep2),
            pl.BlockSpec((V, V), rep2),
            pl.BlockSpec((1, V), rep2),
            pl.BlockSpec((V, V), rep2),
            pl.BlockSpec((1, V), rep2),
        ],
        out_specs=pl.BlockSpec((1, BB, V), lambda i, v: (i, 0, 0)),
        out_shape=jax.ShapeDtypeStruct((nb, BB, V), jnp.float32),
        scratch_shapes=[
            pltpu.VMEM((BB, M), jnp.float32),
            pltpu.VMEM((BB, VH), jnp.float32),
        ],
    )(hidden3, keys4, values4, filled3,
      Wq, bq.reshape(1, K), W1h, W1r, b1.reshape(1, V),
      W2, b2.reshape(1, V), Wo, bo.reshape(1, V))
    return out.reshape(B, V)


# speedup vs baseline: 1.0443x; 1.0194x over previous
import math

import jax
import jax.numpy as jnp
from jax.experimental import pallas as pl

B = 128
M = 1024
K = 128
V = 512

BB = 8


def _episodic_kernel(hidden_ref, keys_ref, values_ref, filled_ref,
                     wq_ref, bq_ref, w1h_ref, w1r_ref, b1_ref,
                     w2_ref, b2_ref, wo_ref, bo_ref, out_ref):
    scale = 1.0 / math.sqrt(K)
    h = hidden_ref[0]
    q = jax.lax.dot_general(h, wq_ref[...], (((1,), (1,)), ((), ())),
                            preferred_element_type=jnp.float32) + bq_ref[...]
    qb = q.astype(jnp.bfloat16)
    scores = jnp.concatenate([
        jax.lax.dot_general(qb[j:j + 1], keys_ref[j].astype(jnp.bfloat16),
                            (((1,), (1,)), ((), ())),
                            preferred_element_type=jnp.float32)
        for j in range(BB)], axis=0)
    slot = jax.lax.broadcasted_iota(jnp.int32, (BB, M), 1)
    valid = slot < filled_ref[0]
    scores = jnp.where(valid, scores * scale, -jnp.inf)
    m = jnp.max(scores, axis=-1, keepdims=True)
    m = jnp.where(jnp.isfinite(m), m, 0.0)
    e = jnp.exp(scores - m)
    s = jnp.sum(e, axis=-1, keepdims=True)
    attn = jnp.where(s > 0.0, e / s, 0.0)
    attnb = attn.astype(jnp.bfloat16)
    retrieved = jnp.concatenate([
        jax.lax.dot_general(attnb[j:j + 1], values_ref[j].astype(jnp.bfloat16),
                            (((1,), (0,)), ((), ())),
                            preferred_element_type=jnp.float32)
        for j in range(BB)], axis=0)
    hb = h.astype(jnp.bfloat16)
    g = (jax.lax.dot_general(hb, w1h_ref[...], (((1,), (1,)), ((), ())),
                             preferred_element_type=jnp.float32)
         + jax.lax.dot_general(retrieved.astype(jnp.bfloat16), w1r_ref[...], (((1,), (1,)), ((), ())),
                               preferred_element_type=jnp.float32)
         + b1_ref[...])
    h1 = g * jax.nn.sigmoid(g)
    gate = jax.nn.sigmoid(
        jax.lax.dot_general(h1.astype(jnp.bfloat16), w2_ref[...], (((1,), (1,)), ((), ())),
                            preferred_element_type=jnp.float32) + b2_ref[...])
    y = h + gate * retrieved
    out_ref[0] = jax.lax.dot_general(y.astype(jnp.bfloat16), wo_ref[...], (((1,), (1,)), ((), ())),
                                     preferred_element_type=jnp.float32) + bo_ref[...]


def kernel(hidden, mem_keys, mem_values, Wq, bq, W1, b1, W2, b2, Wo, bo, filled):
    nsteps = B // BB
    hidden3 = hidden.reshape(nsteps, BB, V)
    filled3 = filled.astype(jnp.int32).reshape(nsteps, BB, 1)
    W1h = W1[:, :V].astype(jnp.bfloat16)
    W1r = W1[:, V:].astype(jnp.bfloat16)
    W2b = W2.astype(jnp.bfloat16)
    Wob = Wo.astype(jnp.bfloat16)
    rep2 = lambda i: (0, 0)

    out = pl.pallas_call(
        _episodic_kernel,
        grid=(nsteps,),
        in_specs=[
            pl.BlockSpec((1, BB, V), lambda i: (i, 0, 0)),
            pl.BlockSpec((BB, M, K), lambda i: (i, 0, 0)),
            pl.BlockSpec((BB, M, V), lambda i: (i, 0, 0)),
            pl.BlockSpec((1, BB, 1), lambda i: (i, 0, 0)),
            pl.BlockSpec((K, V), rep2),
            pl.BlockSpec((1, K), rep2),
            pl.BlockSpec((V, V), rep2),
            pl.BlockSpec((V, V), rep2),
            pl.BlockSpec((1, V), rep2),
            pl.BlockSpec((V, V), rep2),
            pl.BlockSpec((1, V), rep2),
            pl.BlockSpec((V, V), rep2),
            pl.BlockSpec((1, V), rep2),
        ],
        out_specs=pl.BlockSpec((1, BB, V), lambda i: (i, 0, 0)),
        out_shape=jax.ShapeDtypeStruct((nsteps, BB, V), jnp.float32),
    )(hidden3, mem_keys, mem_values, filled3,
      Wq, bq.reshape(1, K), W1h, W1r, b1.reshape(1, V),
      W2b, b2.reshape(1, V), Wob, bo.reshape(1, V))
    return out.reshape(B, V)


# speedup vs baseline: 1.0880x; 1.0418x over previous
import math

import jax
import jax.numpy as jnp
from jax.experimental import pallas as pl

B = 128
M = 1024
K = 128
V = 512

BB = 8


def _episodic_kernel(hidden_ref, keys_ref, values_ref, filled_ref,
                     wq_ref, bq_ref, w1h_ref, w1r_ref, b1_ref,
                     w2_ref, b2_ref, wo_ref, bo_ref, out_ref):
    scale = 1.0 / math.sqrt(K)
    h = hidden_ref[0]
    q = jax.lax.dot_general(h, wq_ref[...], (((1,), (1,)), ((), ())),
                            preferred_element_type=jnp.float32, precision=jax.lax.Precision.DEFAULT) + bq_ref[...]
    scores = jnp.concatenate([
        jax.lax.dot_general(q[j:j + 1], keys_ref[j], (((1,), (1,)), ((), ())),
                            preferred_element_type=jnp.float32, precision=jax.lax.Precision.DEFAULT)
        for j in range(BB)], axis=0)
    slot = jax.lax.broadcasted_iota(jnp.int32, (BB, M), 1)
    valid = slot < filled_ref[0]
    scores = jnp.where(valid, scores * scale, -jnp.inf)
    m = jnp.max(scores, axis=-1, keepdims=True)
    m = jnp.where(jnp.isfinite(m), m, 0.0)
    e = jnp.exp(scores - m)
    s = jnp.sum(e, axis=-1, keepdims=True)
    attn = jnp.where(s > 0.0, e / s, 0.0)
    retrieved = jnp.concatenate([
        jax.lax.dot_general(attn[j:j + 1], values_ref[j], (((1,), (0,)), ((), ())),
                            preferred_element_type=jnp.float32, precision=jax.lax.Precision.DEFAULT)
        for j in range(BB)], axis=0)
    g = (jax.lax.dot_general(h, w1h_ref[...], (((1,), (1,)), ((), ())),
                             preferred_element_type=jnp.float32, precision=jax.lax.Precision.DEFAULT)
         + jax.lax.dot_general(retrieved, w1r_ref[...], (((1,), (1,)), ((), ())),
                               preferred_element_type=jnp.float32, precision=jax.lax.Precision.DEFAULT)
         + b1_ref[...])
    h1 = g * jax.nn.sigmoid(g)
    gate = jax.nn.sigmoid(
        jax.lax.dot_general(h1, w2_ref[...], (((1,), (1,)), ((), ())),
                            preferred_element_type=jnp.float32, precision=jax.lax.Precision.DEFAULT) + b2_ref[...])
    y = h + gate * retrieved
    out_ref[0] = jax.lax.dot_general(y, wo_ref[...], (((1,), (1,)), ((), ())),
                                     preferred_element_type=jnp.float32, precision=jax.lax.Precision.DEFAULT) + bo_ref[...]


def kernel(hidden, mem_keys, mem_values, Wq, bq, W1, b1, W2, b2, Wo, bo, filled):
    nsteps = B // BB
    hidden3 = hidden.reshape(nsteps, BB, V)
    filled3 = filled.astype(jnp.int32).reshape(nsteps, BB, 1)
    W1h = W1[:, :V]
    W1r = W1[:, V:]
    rep2 = lambda i: (0, 0)

    out = pl.pallas_call(
        _episodic_kernel,
        grid=(nsteps,),
        in_specs=[
            pl.BlockSpec((1, BB, V), lambda i: (i, 0, 0)),
            pl.BlockSpec((BB, M, K), lambda i: (i, 0, 0)),
            pl.BlockSpec((BB, M, V), lambda i: (i, 0, 0)),
            pl.BlockSpec((1, BB, 1), lambda i: (i, 0, 0)),
            pl.BlockSpec((K, V), rep2),
            pl.BlockSpec((1, K), rep2),
            pl.BlockSpec((V, V), rep2),
            pl.BlockSpec((V, V), rep2),
            pl.BlockSpec((1, V), rep2),
            pl.BlockSpec((V, V), rep2),
            pl.BlockSpec((1, V), rep2),
            pl.BlockSpec((V, V), rep2),
            pl.BlockSpec((1, V), rep2),
        ],
        out_specs=pl.BlockSpec((1, BB, V), lambda i: (i, 0, 0)),
        out_shape=jax.ShapeDtypeStruct((nsteps, BB, V), jnp.float32),
    )(hidden3, mem_keys, mem_values, filled3,
      Wq, bq.reshape(1, K), W1h, W1r, b1.reshape(1, V),
      W2, b2.reshape(1, V), Wo, bo.reshape(1, V))
    return out.reshape(B, V)
